# baseline (speedup 1.0000x reference)
import functools

import jax
import jax.numpy as jnp
from jax import lax
from jax.experimental import pallas as pl
from jax.experimental.pallas import tpu as pltpu
from jax.experimental.pallas import tpu_sc as plsc

N = 131072
D = 8
NSEG = 512
NROW = 640
BROW = 544
NW = 32
PTS_W = N // NW
TBL = NROW * 16

XC_LO = 1.0000005000002917e-06
XC_HI = 13.815510557964274
LN2 = 0.6931471805599453
_LOGC = (4.342868489e-02, -7.914951135e-02, 1.418487937e-01,
         -3.199195022e-01, 9.618147814e-01, 5.849542865e-01)

_DN2 = (((1,), (0,)), ((), ()))
_PREC = lax.Precision.HIGHEST


@functools.cache
def _get_sc_stats():
    mesh = plsc.VectorSubcoreMesh(core_axis_name="c", subcore_axis_name="s")
    return functools.partial(
        pl.kernel,
        mesh=mesh,
        out_type=jax.ShapeDtypeStruct((NW, TBL), jnp.float32),
        compiler_params=pltpu.CompilerParams(needs_layout_passes=False),
        scratch_types=[
            pltpu.VMEM((D, PTS_W), jnp.float32),
            pltpu.VMEM((PTS_W,), jnp.float32),
            pltpu.VMEM((PTS_W,), jnp.int32),
            pltpu.VMEM((PTS_W,), jnp.int32),
            pltpu.VMEM((PTS_W,), jnp.int32),
            pltpu.VMEM((TBL,), jnp.float32),
        ],
    )(_sc_stats_body)


def _sc_stats_body(et_hbm, mt_hbm, sl_hbm, cl_hbm, bi_hbm, zeros_hbm, out_hbm,
                   e_v, m_v, sl_v, cl_v, bi_v, tbl):
    cid = lax.axis_index("c")
    sub = lax.axis_index("s")
    wid = sub * 2 + cid
    base = wid * PTS_W

    pltpu.sync_copy(et_hbm.at[:, pl.ds(base, PTS_W)], e_v)
    pltpu.sync_copy(mt_hbm.at[pl.ds(base, PTS_W)], m_v)
    pltpu.sync_copy(sl_hbm.at[pl.ds(base, PTS_W)], sl_v)
    pltpu.sync_copy(cl_hbm.at[pl.ds(base, PTS_W)], cl_v)
    pltpu.sync_copy(bi_hbm.at[pl.ds(base, PTS_W)], bi_v)
    pltpu.sync_copy(zeros_hbm, tbl)

    ones = jnp.ones((16,), jnp.float32)

    def body(i, carry):
        off = i * 16
        sl = sl_v[pl.ds(off, 16)]
        cl = cl_v[pl.ds(off, 16)]
        bi = bi_v[pl.ds(off, 16)]
        seg = bi * 128 + sl * 32 + cl
        seg = jnp.where(sl < 4, seg, 560)
        addr = seg * 16
        for q in range(D):
            plsc.addupdate_scatter(tbl, [addr + q], e_v[q, pl.ds(off, 16)])
        plsc.addupdate_scatter(tbl, [addr + 8], m_v[pl.ds(off, 16)])
        plsc.addupdate_scatter(tbl, [addr + 9], ones)
        plsc.addupdate_scatter(tbl, [(bi + BROW) * 16 + 9], ones)
        return carry

    lax.fori_loop(0, PTS_W // 16, body, 0)

    pltpu.sync_copy(tbl, out_hbm.at[wid])


def _finalize_kernel(tables_ref, stats_out, cmat_out, aux_out):
    acc = tables_ref[0]
    for w in range(1, NW):
        acc = acc + tables_ref[w]
    stats_out[...] = acc
    cnt = acc[0:NSEG, 9:10]
    inv_cnt = 1.0 / jnp.maximum(cnt, 1.0)
    cmat = acc[0:NSEG, 0:D] * inv_cnt
    cmat_out[...] = cmat
    sigma = acc[0:NSEG, 8:9] * inv_cnt
    inv_den = 1.0 / (2.0 * sigma * sigma + 1e-8)
    cnorm2 = jnp.sum(cmat * cmat, axis=1, keepdims=True)
    aux_out[...] = jnp.concatenate(
        [inv_den, cnorm2, sigma, jnp.zeros_like(sigma)], axis=1)


@functools.cache
def _get_sc_loss():
    mesh = plsc.VectorSubcoreMesh(core_axis_name="c", subcore_axis_name="s")
    return functools.partial(
        pl.kernel,
        mesh=mesh,
        out_type=jax.ShapeDtypeStruct((NW, 1024), jnp.float32),
        compiler_params=pltpu.CompilerParams(needs_layout_passes=False),
        scratch_types=[
            pltpu.VMEM((D, PTS_W), jnp.float32),
            pltpu.VMEM((PTS_W,), jnp.float32),
            pltpu.VMEM((PTS_W,), jnp.int32),
            pltpu.VMEM((PTS_W,), jnp.int32),
            pltpu.VMEM((PTS_W,), jnp.int32),
            pltpu.VMEM((NSEG * D,), jnp.float32),
            pltpu.VMEM((NSEG * 4,), jnp.float32),
            pltpu.VMEM((1024,), jnp.float32),
        ],
    )(_sc_loss_body)


def _sc_loss_body(et_hbm, mt_hbm, sl_hbm, cl_hbm, bi_hbm, cmat_hbm, aux_hbm,
                  zeros_hbm, out_hbm,
                  e_v, m_v, sl_v, cl_v, bi_v, cent_v, aux_v, ptbl):
    cid = lax.axis_index("c")
    sub = lax.axis_index("s")
    wid = sub * 2 + cid
    base = wid * PTS_W

    pltpu.sync_copy(et_hbm.at[:, pl.ds(base, PTS_W)], e_v)
    pltpu.sync_copy(mt_hbm.at[pl.ds(base, PTS_W)], m_v)
    pltpu.sync_copy(sl_hbm.at[pl.ds(base, PTS_W)], sl_v)
    pltpu.sync_copy(cl_hbm.at[pl.ds(base, PTS_W)], cl_v)
    pltpu.sync_copy(bi_hbm.at[pl.ds(base, PTS_W)], bi_v)
    pltpu.sync_copy(cmat_hbm, cent_v)
    pltpu.sync_copy(aux_hbm, aux_v)
    pltpu.sync_copy(zeros_hbm, ptbl)

    zeros16 = jnp.zeros((16,), jnp.int32)

    def body(i, carry):
        off = i * 16
        sl = sl_v[pl.ds(off, 16)]
        cl = cl_v[pl.ds(off, 16)]
        bi = bi_v[pl.ds(off, 16)]
        m16 = m_v[pl.ds(off, 16)]
        valid = sl < 4
        gbase = jnp.where(valid, (bi * 4 + sl) * 32, 0)
        ed = [e_v[d, pl.ds(off, 16)] for d in range(D)]
        enorm2 = ed[0] * ed[0]
        for d in range(1, D):
            enorm2 = enorm2 + ed[d] * ed[d]

        sid_own = gbase + cl
        sig_own = plsc.load_gather(aux_v, [sid_own * 4 + 2])
        dmm = m16 - sig_own
        plsc.addupdate_scatter(ptbl, [sid_own + 512], dmm * dmm, mask=valid)

        for c in range(32):
            idxc = gbase + c
            adr = idxc * D
            dot = ed[0] * plsc.load_gather(cent_v, [adr])
            for d in range(1, D):
                dot = dot + ed[d] * plsc.load_gather(cent_v, [adr + d])
            invd = plsc.load_gather(aux_v, [idxc * 4])
            cn2 = plsc.load_gather(aux_v, [idxc * 4 + 1])
            x = (cn2 - 2.0 * dot + enorm2) * invd
            xc = jnp.clip(x, XC_LO, XC_HI)
            p = jnp.exp(-xc)
            z = 1.0 - p
            bits = plsc.bitcast(z, jnp.int32)
            ebits = lax.shift_right_logical(bits, 23) - 127
            mant = plsc.bitcast(
                jnp.bitwise_or(jnp.bitwise_and(bits, 0x007FFFFF), 0x3F800000),
                jnp.float32)
            t = mant - 1.5
            poly = jnp.float32(_LOGC[0])
            for cc in _LOGC[1:]:
                poly = poly * t + cc
            lnz = (poly + ebits.astype(jnp.float32)) * LN2
            term = jnp.where(cl == c, xc, -lnz)
            plsc.addupdate_scatter(ptbl, [idxc], term, mask=valid)
        return carry

    lax.fori_loop(0, PTS_W // 16, body, 0)

    pltpu.sync_copy(ptbl, out_hbm.at[wid])


def _combine_kernel(parts_ref, stats_ref, out_ref):
    acc = parts_ref[0]
    for w in range(1, NW):
        acc = acc + parts_ref[w]
    acc = acc.reshape(1, 1024)
    bce_row = acc[:, 0:NSEG]
    sm_row = acc[:, NSEG:1024]

    ii = lax.broadcasted_iota(jnp.int32, (NSEG, NSEG), 0)
    jj = lax.broadcasted_iota(jnp.int32, (NSEG, NSEG), 1)
    ident = (ii == jj).astype(jnp.float32)
    tr = (((1,), (1,)), ((), ()))

    def tcol(row):
        return lax.dot_general(ident, row, tr,
                               preferred_element_type=jnp.float32,
                               precision=_PREC)

    bce = tcol(bce_row)
    sm = tcol(sm_row)

    cnt = stats_ref[0:NSEG, 9:10]

    gi = lax.broadcasted_iota(jnp.int32, (16, NSEG), 0)
    si = lax.broadcasted_iota(jnp.int32, (16, NSEG), 1)
    m1 = ((si // 32) == gi).astype(jnp.float32)
    bi4 = lax.broadcasted_iota(jnp.int32, (4, 16), 0)
    gi16 = lax.broadcasted_iota(jnp.int32, (4, 16), 1)
    m2 = ((gi16 // 4) == bi4).astype(jnp.float32)

    def gdot(mat, vec):
        return lax.dot_general(mat, vec, _DN2,
                               preferred_element_type=jnp.float32,
                               precision=_PREC)

    present = (cnt > 0.0).astype(jnp.float32)
    n_sel = gdot(m1, cnt)
    npres = gdot(m1, present)
    bce_g = gdot(m1, present * bce)
    sm_g = gdot(m1, sm)
    n_sel_safe = jnp.maximum(n_sel, 1.0)
    npres_safe = jnp.maximum(npres, 1.0)
    ml = bce_g / n_sel_safe / npres_safe
    sml = sm_g / npres_safe
    s_present = (n_sel > 0.0).astype(jnp.float32)
    contrib = s_present * (ml + sml)
    cls_sum = gdot(m2, contrib)
    cls_cnt = gdot(m2, s_present)
    batch_loss = cls_sum / jnp.maximum(cls_cnt, 1.0)
    bcnt = stats_ref[BROW:BROW + 4, 9:10]
    b_present = (bcnt > 0.0).astype(jnp.float32)
    num = jnp.sum(b_present * batch_loss, keepdims=True)
    den = jnp.maximum(jnp.sum(b_present, keepdims=True), 1.0)
    out_ref[...] = num / den


@jax.jit
def kernel(embeddings, margins, slabels, clabels, batch_idx):
    et = embeddings.T
    mt = margins.reshape(N)
    sl32 = slabels.astype(jnp.int32)
    cl32 = clabels.astype(jnp.int32)
    bi32 = batch_idx.astype(jnp.int32)
    zeros_t = jnp.zeros((TBL,), jnp.float32)
    zeros_p = jnp.zeros((1024,), jnp.float32)

    tables = _get_sc_stats()(et, mt, sl32, cl32, bi32, zeros_t)
    tables = tables.reshape(NW, NROW, 16)

    stats, cmat, aux = pl.pallas_call(
        _finalize_kernel,
        out_shape=[jax.ShapeDtypeStruct((NROW, 16), jnp.float32),
                   jax.ShapeDtypeStruct((NSEG, D), jnp.float32),
                   jax.ShapeDtypeStruct((NSEG, 4), jnp.float32)],
    )(tables)

    parts = _get_sc_loss()(et, mt, sl32, cl32, bi32,
                           cmat.reshape(NSEG * D), aux.reshape(NSEG * 4),
                           zeros_p)

    out = pl.pallas_call(
        _combine_kernel,
        out_shape=jax.ShapeDtypeStruct((1, 1), jnp.float32),
    )(parts, stats)

    return out[0, 0]
